# baseline (speedup 1.0000x reference)
import functools

import jax
import jax.numpy as jnp
from jax import lax
from jax.experimental import pallas as pl
from jax.experimental.pallas import tpu as pltpu
from jax.experimental.pallas import tpu_sc as plsc

_NUM_CORES = 2
_NUM_SUBCORES = 16
_NW = _NUM_CORES * _NUM_SUBCORES

_CHUNK = 128
_NBUF = 8


@functools.partial(jax.jit, static_argnames=("n_rows", "dim"))
def _sc_gather(table, idx, *, n_rows, dim):
    b_per_w = n_rows // _NW
    chunks = b_per_w // _CHUNK
    nblocks = chunks // _NBUF

    mesh = plsc.VectorSubcoreMesh(
        core_axis_name="c", subcore_axis_name="s", num_cores=_NUM_CORES
    )

    @functools.partial(
        pl.kernel,
        mesh=mesh,
        compiler_params=pltpu.CompilerParams(use_tc_tiling_on_sc=False),
        out_type=jax.ShapeDtypeStruct((n_rows, dim), table.dtype),
        scratch_types=(
            [pltpu.VMEM((b_per_w,), jnp.int32)]
            + [pltpu.VMEM((_CHUNK, dim), table.dtype) for _ in range(_NBUF)]
            + [pltpu.SemaphoreType.DMA for _ in range(_NBUF)]
            + [pltpu.SemaphoreType.DMA]
        ),
    )
    def run(table_hbm, idx_hbm, out_hbm, idx_v, *rest):
        bufs = rest[:_NBUF]
        gsems = rest[_NBUF : 2 * _NBUF]
        osem = rest[2 * _NBUF]

        wid = lax.axis_index("s") * _NUM_CORES + lax.axis_index("c")
        base = wid * b_per_w
        pltpu.sync_copy(idx_hbm.at[pl.ds(base, b_per_w)], idx_v)

        def block(o, carry):
            g0 = o * (_NBUF * _CHUNK)
            for b in range(_NBUF):
                off = pl.multiple_of(g0 + b * _CHUNK, 8)
                pltpu.make_async_copy(
                    table_hbm.at[idx_v.at[pl.ds(off, _CHUNK)]],
                    bufs[b],
                    gsems[b],
                ).start()
            for b in range(_NBUF):
                off = pl.multiple_of(g0 + b * _CHUNK, 8)
                pltpu.make_async_copy(
                    table_hbm.at[idx_v.at[pl.ds(off, _CHUNK)]],
                    bufs[b],
                    gsems[b],
                ).wait()
                pltpu.make_async_copy(
                    bufs[b],
                    out_hbm.at[pl.ds(pl.multiple_of(base + off, 8), _CHUNK)],
                    osem,
                ).start()
            for b in range(_NBUF):
                off = pl.multiple_of(g0 + b * _CHUNK, 8)
                pltpu.make_async_copy(
                    bufs[b],
                    out_hbm.at[pl.ds(pl.multiple_of(base + off, 8), _CHUNK)],
                    osem,
                ).wait()
            return carry

        lax.fori_loop(0, nblocks, block, 0)

    return run(table, idx)


def kernel(x, lengths, table):
    del lengths
    batch, hist = x.shape
    dim = table.shape[1]
    idx = x.reshape(-1).astype(jnp.int32)
    out = _sc_gather(table, idx, n_rows=batch * hist, dim=dim)
    return out.reshape(batch, hist, dim)
